# baseline (speedup 1.0000x reference)
import functools

import jax
import jax.numpy as jnp
from jax import lax
from jax.experimental import pallas as pl
from jax.experimental.pallas import tpu as pltpu
from jax.experimental.pallas import tpu_sc as plsc

N = 10000
IN_DIM = 128
HID = 16
OUT_DIM = 128
NC, NS = 2, 16
NW = NC * NS
N_PAD = 10240
TRASH = N
CHUNK = 128
NCH = 80
E_PAD = NW * NCH * CHUNK
GRP = 2
IDXW = GRP * CHUNK
NGRP = NCH // GRP
NBUF = 4
RDIST = 2
RPT = N_PAD // NS


def _vector_mesh():
    return plsc.VectorSubcoreMesh(core_axis_name="c", subcore_axis_name="s")


_SC_PARAMS = pltpu.CompilerParams(use_tc_tiling_on_sc=False,
                                  needs_layout_passes=False)


def _rsqrt16(d):
    i = plsc.bitcast(d, jnp.int32)
    i = jnp.full((HID,), 0x5F3759DF, jnp.int32) - lax.shift_right_logical(i, 1)
    y = plsc.bitcast(i, jnp.float32)
    for _ in range(3):
        y = y * (1.5 - 0.5 * d * y * y)
    return y


def _sc_degree(dst3):
    @functools.partial(
        pl.kernel,
        out_type=jax.ShapeDtypeStruct((NC, N_PAD, HID), jnp.float32),
        mesh=_vector_mesh(),
        compiler_params=_SC_PARAMS,
        scratch_types=[
            pltpu.VMEM((NGRP, IDXW), jnp.int32),
            pltpu.VMEM((IDXW, HID), jnp.float32),
            pltpu.VMEM((RPT, HID), jnp.float32),
            pltpu.VMEM_SHARED((N_PAD, HID), jnp.float32),
            pltpu.SemaphoreType.DMA,
        ],
    )
    def deg_kernel(dst_hbm, out_hbm, didx, ones_buf, zbuf, acc, ssem):
        cid = lax.axis_index("c")
        sid = lax.axis_index("s")
        wid = cid * NS + sid
        r0 = sid * RPT

        pltpu.sync_copy(dst_hbm.at[wid], didx)

        @pl.loop(0, RPT)
        def _(i):
            zbuf[i] = jnp.zeros((HID,), jnp.float32)

        @pl.loop(0, IDXW)
        def _(i):
            ones_buf[i] = jnp.ones((HID,), jnp.float32)

        pltpu.sync_copy(zbuf, acc.at[pl.ds(r0, RPT)])
        plsc.subcore_barrier()

        @pl.loop(0, NGRP)
        def _(c):
            pltpu.async_copy(ones_buf, acc.at[didx.at[c]], ssem, add=True)

        @pl.loop(0, NGRP)
        def _(c):
            pltpu.make_async_copy(ones_buf, acc.at[didx.at[c]], ssem).wait()

        plsc.subcore_barrier()
        pltpu.sync_copy(acc.at[pl.ds(r0, RPT)],
                        out_hbm.at[cid, pl.ds(r0, RPT)])

    return deg_kernel(dst3)


def _agg_main_loop(table, sidx, didx, rows, acc, gsems, ssems):
    for j in range(NBUF):
        pltpu.async_copy(table.at[sidx.at[j]], rows.at[j], gsems.at[j])

    @pl.loop(0, NGRP, step=NBUF)
    def _(g):
        for j in range(NBUF):
            c = g + j
            pltpu.make_async_copy(table.at[sidx.at[c]], rows.at[j],
                                  gsems.at[j]).wait()
            pltpu.async_copy(rows.at[j], acc.at[didx.at[c]],
                             ssems.at[j], add=True)
            cr = c - RDIST
            jr = (j - RDIST) % NBUF

            @pl.when(jnp.logical_and(cr >= 0, cr + NBUF < NGRP))
            def _():
                pltpu.make_async_copy(rows.at[jr], acc.at[didx.at[cr]],
                                      ssems.at[jr]).wait()
                pltpu.async_copy(table.at[sidx.at[cr + NBUF]],
                                 rows.at[jr], gsems.at[jr])

    for j in range(NBUF):
        c = NGRP - NBUF + j
        pltpu.make_async_copy(rows.at[j], acc.at[didx.at[c]],
                              ssems.at[j]).wait()


def _sc_agg1(h1, deg2, src3, dst3):
    @functools.partial(
        pl.kernel,
        out_type=jax.ShapeDtypeStruct((NC, N_PAD, HID), jnp.float32),
        mesh=_vector_mesh(),
        compiler_params=_SC_PARAMS,
        scratch_types=[
            pltpu.VMEM((NGRP, IDXW), jnp.int32),
            pltpu.VMEM((NGRP, IDXW), jnp.int32),
            pltpu.VMEM((NBUF, IDXW, HID), jnp.float32),
            pltpu.VMEM((RPT, HID), jnp.float32),
            pltpu.VMEM((RPT, HID), jnp.float32),
            pltpu.VMEM((RPT, HID), jnp.float32),
            pltpu.VMEM((RPT, HID), jnp.float32),
            pltpu.VMEM_SHARED((N_PAD, HID), jnp.float32),
            pltpu.VMEM_SHARED((N_PAD, HID), jnp.float32),
            pltpu.SemaphoreType.DMA((NBUF,)),
            pltpu.SemaphoreType.DMA((NBUF,)),
            pltpu.SemaphoreType.DMA,
        ],
    )
    def agg1_kernel(h1_hbm, deg_hbm, src_hbm, dst_hbm, out_hbm,
                    sidx, didx, rows, hbuf, d0buf, d1buf, pbuf,
                    table, acc, gsems, ssems, psem):
        cid = lax.axis_index("c")
        sid = lax.axis_index("s")
        wid = cid * NS + sid
        r0 = sid * RPT
        sl = pl.ds(r0, RPT)

        pltpu.async_copy(src_hbm.at[wid], sidx, psem)
        pltpu.async_copy(dst_hbm.at[wid], didx, psem)
        pltpu.async_copy(h1_hbm.at[sl], hbuf, psem)
        pltpu.async_copy(deg_hbm.at[0, sl], d0buf, psem)
        pltpu.async_copy(deg_hbm.at[1, sl], d1buf, psem)
        pltpu.make_async_copy(src_hbm.at[wid], sidx, psem).wait()
        pltpu.make_async_copy(dst_hbm.at[wid], didx, psem).wait()
        pltpu.make_async_copy(h1_hbm.at[sl], hbuf, psem).wait()
        pltpu.make_async_copy(deg_hbm.at[0, sl], d0buf, psem).wait()
        pltpu.make_async_copy(deg_hbm.at[1, sl], d1buf, psem).wait()

        @pl.loop(0, RPT)
        def _(i):
            d = d0buf[i] + d1buf[i] + 1.0
            pbuf[i] = _rsqrt16(d) * hbuf[i]

        pltpu.sync_copy(pbuf, table.at[sl])

        @pl.loop(0, RPT)
        def _(i):
            pbuf[i] = jnp.zeros((HID,), jnp.float32)

        pltpu.sync_copy(pbuf, acc.at[sl])
        plsc.subcore_barrier()

        _agg_main_loop(table, sidx, didx, rows, acc, gsems, ssems)

        plsc.subcore_barrier()
        pltpu.sync_copy(acc.at[sl], out_hbm.at[cid, sl])

    return agg1_kernel(h1, deg2, src3, dst3)


def _sc_agg2(acc1, deg2, h1, b1r, src3, dst3):
    @functools.partial(
        pl.kernel,
        out_type=(
            jax.ShapeDtypeStruct((NC, N_PAD, HID), jnp.float32),
            jax.ShapeDtypeStruct((N_PAD, HID), jnp.float32),
        ),
        mesh=_vector_mesh(),
        compiler_params=_SC_PARAMS,
        scratch_types=[
            pltpu.VMEM((NGRP, IDXW), jnp.int32),
            pltpu.VMEM((NGRP, IDXW), jnp.int32),
            pltpu.VMEM((NBUF, IDXW, HID), jnp.float32),
            pltpu.VMEM((RPT, HID), jnp.float32),
            pltpu.VMEM((RPT, HID), jnp.float32),
            pltpu.VMEM((RPT, HID), jnp.float32),
            pltpu.VMEM((RPT, HID), jnp.float32),
            pltpu.VMEM((RPT, HID), jnp.float32),
            pltpu.VMEM((RPT, HID), jnp.float32),
            pltpu.VMEM((RPT, HID), jnp.float32),
            pltpu.VMEM((HID,), jnp.float32),
            pltpu.VMEM_SHARED((N_PAD, HID), jnp.float32),
            pltpu.VMEM_SHARED((N_PAD, HID), jnp.float32),
            pltpu.SemaphoreType.DMA((NBUF,)),
            pltpu.SemaphoreType.DMA((NBUF,)),
            pltpu.SemaphoreType.DMA,
        ],
    )
    def agg2_kernel(acc1_hbm, deg_hbm, h1_hbm, b1_hbm, src_hbm, dst_hbm,
                    out_hbm, q_hbm,
                    sidx, didx, rows, hbuf, d0buf, d1buf, a0buf, a1buf,
                    pbuf, qbuf, bbuf, table, acc, gsems, ssems, psem):
        cid = lax.axis_index("c")
        sid = lax.axis_index("s")
        wid = cid * NS + sid
        r0 = sid * RPT
        sl = pl.ds(r0, RPT)

        pltpu.async_copy(src_hbm.at[wid], sidx, psem)
        pltpu.async_copy(dst_hbm.at[wid], didx, psem)
        pltpu.async_copy(h1_hbm.at[sl], hbuf, psem)
        pltpu.async_copy(deg_hbm.at[0, sl], d0buf, psem)
        pltpu.async_copy(deg_hbm.at[1, sl], d1buf, psem)
        pltpu.async_copy(acc1_hbm.at[0, sl], a0buf, psem)
        pltpu.async_copy(acc1_hbm.at[1, sl], a1buf, psem)
        pltpu.async_copy(b1_hbm.at[0], bbuf, psem)
        pltpu.make_async_copy(src_hbm.at[wid], sidx, psem).wait()
        pltpu.make_async_copy(dst_hbm.at[wid], didx, psem).wait()
        pltpu.make_async_copy(h1_hbm.at[sl], hbuf, psem).wait()
        pltpu.make_async_copy(deg_hbm.at[0, sl], d0buf, psem).wait()
        pltpu.make_async_copy(deg_hbm.at[1, sl], d1buf, psem).wait()
        pltpu.make_async_copy(acc1_hbm.at[0, sl], a0buf, psem).wait()
        pltpu.make_async_copy(acc1_hbm.at[1, sl], a1buf, psem).wait()
        pltpu.make_async_copy(b1_hbm.at[0], bbuf, psem).wait()

        bb = bbuf[...]

        @pl.loop(0, RPT)
        def _(i):
            d = d0buf[i] + d1buf[i] + 1.0
            y = _rsqrt16(d)
            y2 = y * y
            s = y * (a0buf[i] + a1buf[i]) + y2 * hbuf[i] + bb
            z = jnp.maximum(s, 0.0)
            pbuf[i] = y * z
            qbuf[i] = y2 * z

        pltpu.sync_copy(pbuf, table.at[sl])

        @pl.when(cid == 0)
        def _():
            pltpu.sync_copy(qbuf, q_hbm.at[sl])

        @pl.loop(0, RPT)
        def _(i):
            pbuf[i] = jnp.zeros((HID,), jnp.float32)

        pltpu.sync_copy(pbuf, acc.at[sl])
        plsc.subcore_barrier()

        _agg_main_loop(table, sidx, didx, rows, acc, gsems, ssems)

        plsc.subcore_barrier()
        pltpu.sync_copy(acc.at[sl], out_hbm.at[cid, sl])

    return agg2_kernel(acc1, deg2, h1, b1r, src3, dst3)


def _tc_matmul1(x_p, w1t):
    def body(x_ref, w1t_ref, h1_ref):
        h1_ref[...] = jnp.dot(x_ref[...], w1t_ref[...],
                              preferred_element_type=jnp.float32)

    return pl.pallas_call(
        body,
        out_shape=jax.ShapeDtypeStruct((N_PAD, HID), jnp.float32),
    )(x_p, w1t)


def _tc_out(acc2, q, deg2, w2t, b2r):
    def body(agg_ref, q_ref, deg_ref, w2t_ref, b2_ref, out_ref):
        dis = lax.rsqrt(deg_ref[0] + deg_ref[1] + 1.0)
        m = dis * (agg_ref[0] + agg_ref[1]) + q_ref[...]
        out = jnp.dot(m, w2t_ref[...],
                      preferred_element_type=jnp.float32) + b2_ref[...]
        out_ref[...] = jnp.maximum(out, 0.0)

    return pl.pallas_call(
        body,
        out_shape=jax.ShapeDtypeStruct((N_PAD, OUT_DIM), jnp.float32),
    )(acc2, q, deg2, w2t, b2r)


def kernel(x, edge_index, W1, b1, W2, b2):
    ei = edge_index.astype(jnp.int32)
    e = ei.shape[1]
    pad = TRASH + jnp.arange(E_PAD - e, dtype=jnp.int32) % (N_PAD - N)
    src3 = jnp.concatenate([ei[0], pad]).reshape(NW, NGRP, IDXW)
    dst3 = jnp.concatenate([ei[1], pad]).reshape(NW, NGRP, IDXW)
    x_p = jnp.pad(x, ((0, N_PAD - N), (0, 0)))
    w1t = W1.T
    w2t = W2.T

    deg2 = _sc_degree(dst3)
    h1 = _tc_matmul1(x_p, w1t)
    acc1 = _sc_agg1(h1, deg2, src3, dst3)
    acc2, q = _sc_agg2(acc1, deg2, h1, b1.reshape(1, HID), src3, dst3)
    out = _tc_out(acc2, q, deg2, w2t, b2.reshape(1, OUT_DIM))
    return out[:N]


# speedup vs baseline: 1.0108x; 1.0108x over previous
import functools

import jax
import jax.numpy as jnp
from jax import lax
from jax.experimental import pallas as pl
from jax.experimental.pallas import tpu as pltpu
from jax.experimental.pallas import tpu_sc as plsc

N = 10000
IN_DIM = 128
HID = 16
OUT_DIM = 128
NC, NS = 2, 16
NW = NC * NS
N_PAD = 10240
TRASH = N
CHUNK = 128
NCH = 80
E_PAD = NW * NCH * CHUNK
NBUF = 8
RDIST = 4
RPT = N_PAD // NS


def _vector_mesh():
    return plsc.VectorSubcoreMesh(core_axis_name="c", subcore_axis_name="s")


_SC_PARAMS = pltpu.CompilerParams(use_tc_tiling_on_sc=False,
                                  needs_layout_passes=False)


def _rsqrt16(d):
    i = plsc.bitcast(d, jnp.int32)
    i = jnp.full((HID,), 0x5F3759DF, jnp.int32) - lax.shift_right_logical(i, 1)
    y = plsc.bitcast(i, jnp.float32)
    for _ in range(3):
        y = y * (1.5 - 0.5 * d * y * y)
    return y


def _sc_degree(dst3):
    @functools.partial(
        pl.kernel,
        out_type=jax.ShapeDtypeStruct((NC, N_PAD, HID), jnp.float32),
        mesh=_vector_mesh(),
        compiler_params=_SC_PARAMS,
        scratch_types=[
            pltpu.VMEM((NCH, CHUNK), jnp.int32),
            pltpu.VMEM((CHUNK, HID), jnp.float32),
            pltpu.VMEM((RPT, HID), jnp.float32),
            pltpu.VMEM_SHARED((N_PAD, HID), jnp.float32),
            pltpu.SemaphoreType.DMA,
        ],
    )
    def deg_kernel(dst_hbm, out_hbm, didx, ones_buf, zbuf, acc, ssem):
        cid = lax.axis_index("c")
        sid = lax.axis_index("s")
        wid = cid * NS + sid
        r0 = sid * RPT

        pltpu.sync_copy(dst_hbm.at[wid], didx)

        @pl.loop(0, RPT)
        def _(i):
            zbuf[i] = jnp.zeros((HID,), jnp.float32)

        @pl.loop(0, CHUNK)
        def _(i):
            ones_buf[i] = jnp.ones((HID,), jnp.float32)

        pltpu.sync_copy(zbuf, acc.at[pl.ds(r0, RPT)])
        plsc.subcore_barrier()

        @pl.loop(0, NCH)
        def _(c):
            pltpu.async_copy(ones_buf, acc.at[didx.at[c]], ssem, add=True)

        @pl.loop(0, NCH)
        def _(c):
            pltpu.make_async_copy(ones_buf, acc.at[didx.at[c]], ssem).wait()

        plsc.subcore_barrier()
        pltpu.sync_copy(acc.at[pl.ds(r0, RPT)],
                        out_hbm.at[cid, pl.ds(r0, RPT)])

    return deg_kernel(dst3)


def _agg_main_loop(table, sidx, didx, rows, acc, gsems, ssems):
    for j in range(NBUF):
        pltpu.async_copy(table.at[sidx.at[j]], rows.at[j], gsems.at[j])

    @pl.loop(0, NCH, step=NBUF)
    def _(g):
        for j in range(NBUF):
            c = g + j
            pltpu.make_async_copy(table.at[sidx.at[c]], rows.at[j],
                                  gsems.at[j]).wait()
            pltpu.async_copy(rows.at[j], acc.at[didx.at[c]],
                             ssems.at[j], add=True)
            cr = c - RDIST
            jr = (j - RDIST) % NBUF

            @pl.when(jnp.logical_and(cr >= 0, cr + NBUF < NCH))
            def _():
                pltpu.make_async_copy(rows.at[jr], acc.at[didx.at[cr]],
                                      ssems.at[jr]).wait()
                pltpu.async_copy(table.at[sidx.at[cr + NBUF]],
                                 rows.at[jr], gsems.at[jr])

    for j in range(NBUF):
        c = NCH - NBUF + j
        pltpu.make_async_copy(rows.at[j], acc.at[didx.at[c]],
                              ssems.at[j]).wait()


def _sc_agg1(h1, deg2, src3, dst3):
    @functools.partial(
        pl.kernel,
        out_type=jax.ShapeDtypeStruct((NC, N_PAD, HID), jnp.float32),
        mesh=_vector_mesh(),
        compiler_params=_SC_PARAMS,
        scratch_types=[
            pltpu.VMEM((NCH, CHUNK), jnp.int32),
            pltpu.VMEM((NCH, CHUNK), jnp.int32),
            pltpu.VMEM((NBUF, CHUNK, HID), jnp.float32),
            pltpu.VMEM((RPT, HID), jnp.float32),
            pltpu.VMEM((RPT, HID), jnp.float32),
            pltpu.VMEM((RPT, HID), jnp.float32),
            pltpu.VMEM((RPT, HID), jnp.float32),
            pltpu.VMEM_SHARED((N_PAD, HID), jnp.float32),
            pltpu.VMEM_SHARED((N_PAD, HID), jnp.float32),
            pltpu.SemaphoreType.DMA((NBUF,)),
            pltpu.SemaphoreType.DMA((NBUF,)),
            pltpu.SemaphoreType.DMA,
        ],
    )
    def agg1_kernel(h1_hbm, deg_hbm, src_hbm, dst_hbm, out_hbm,
                    sidx, didx, rows, hbuf, d0buf, d1buf, pbuf,
                    table, acc, gsems, ssems, psem):
        cid = lax.axis_index("c")
        sid = lax.axis_index("s")
        wid = cid * NS + sid
        r0 = sid * RPT
        sl = pl.ds(r0, RPT)

        pltpu.async_copy(src_hbm.at[wid], sidx, psem)
        pltpu.async_copy(dst_hbm.at[wid], didx, psem)
        pltpu.async_copy(h1_hbm.at[sl], hbuf, psem)
        pltpu.async_copy(deg_hbm.at[0, sl], d0buf, psem)
        pltpu.async_copy(deg_hbm.at[1, sl], d1buf, psem)
        pltpu.make_async_copy(src_hbm.at[wid], sidx, psem).wait()
        pltpu.make_async_copy(dst_hbm.at[wid], didx, psem).wait()
        pltpu.make_async_copy(h1_hbm.at[sl], hbuf, psem).wait()
        pltpu.make_async_copy(deg_hbm.at[0, sl], d0buf, psem).wait()
        pltpu.make_async_copy(deg_hbm.at[1, sl], d1buf, psem).wait()

        @pl.loop(0, RPT)
        def _(i):
            d = d0buf[i] + d1buf[i] + 1.0
            pbuf[i] = _rsqrt16(d) * hbuf[i]

        pltpu.sync_copy(pbuf, table.at[sl])

        @pl.loop(0, RPT)
        def _(i):
            pbuf[i] = jnp.zeros((HID,), jnp.float32)

        pltpu.sync_copy(pbuf, acc.at[sl])
        plsc.subcore_barrier()

        _agg_main_loop(table, sidx, didx, rows, acc, gsems, ssems)

        plsc.subcore_barrier()
        pltpu.sync_copy(acc.at[sl], out_hbm.at[cid, sl])

    return agg1_kernel(h1, deg2, src3, dst3)


def _sc_agg2(acc1, deg2, h1, b1r, src3, dst3):
    @functools.partial(
        pl.kernel,
        out_type=(
            jax.ShapeDtypeStruct((NC, N_PAD, HID), jnp.float32),
            jax.ShapeDtypeStruct((N_PAD, HID), jnp.float32),
        ),
        mesh=_vector_mesh(),
        compiler_params=_SC_PARAMS,
        scratch_types=[
            pltpu.VMEM((NCH, CHUNK), jnp.int32),
            pltpu.VMEM((NCH, CHUNK), jnp.int32),
            pltpu.VMEM((NBUF, CHUNK, HID), jnp.float32),
            pltpu.VMEM((RPT, HID), jnp.float32),
            pltpu.VMEM((RPT, HID), jnp.float32),
            pltpu.VMEM((RPT, HID), jnp.float32),
            pltpu.VMEM((RPT, HID), jnp.float32),
            pltpu.VMEM((RPT, HID), jnp.float32),
            pltpu.VMEM((RPT, HID), jnp.float32),
            pltpu.VMEM((RPT, HID), jnp.float32),
            pltpu.VMEM((HID,), jnp.float32),
            pltpu.VMEM_SHARED((N_PAD, HID), jnp.float32),
            pltpu.VMEM_SHARED((N_PAD, HID), jnp.float32),
            pltpu.SemaphoreType.DMA((NBUF,)),
            pltpu.SemaphoreType.DMA((NBUF,)),
            pltpu.SemaphoreType.DMA,
        ],
    )
    def agg2_kernel(acc1_hbm, deg_hbm, h1_hbm, b1_hbm, src_hbm, dst_hbm,
                    out_hbm, q_hbm,
                    sidx, didx, rows, hbuf, d0buf, d1buf, a0buf, a1buf,
                    pbuf, qbuf, bbuf, table, acc, gsems, ssems, psem):
        cid = lax.axis_index("c")
        sid = lax.axis_index("s")
        wid = cid * NS + sid
        r0 = sid * RPT
        sl = pl.ds(r0, RPT)

        pltpu.async_copy(src_hbm.at[wid], sidx, psem)
        pltpu.async_copy(dst_hbm.at[wid], didx, psem)
        pltpu.async_copy(h1_hbm.at[sl], hbuf, psem)
        pltpu.async_copy(deg_hbm.at[0, sl], d0buf, psem)
        pltpu.async_copy(deg_hbm.at[1, sl], d1buf, psem)
        pltpu.async_copy(acc1_hbm.at[0, sl], a0buf, psem)
        pltpu.async_copy(acc1_hbm.at[1, sl], a1buf, psem)
        pltpu.async_copy(b1_hbm.at[0], bbuf, psem)
        pltpu.make_async_copy(src_hbm.at[wid], sidx, psem).wait()
        pltpu.make_async_copy(dst_hbm.at[wid], didx, psem).wait()
        pltpu.make_async_copy(h1_hbm.at[sl], hbuf, psem).wait()
        pltpu.make_async_copy(deg_hbm.at[0, sl], d0buf, psem).wait()
        pltpu.make_async_copy(deg_hbm.at[1, sl], d1buf, psem).wait()
        pltpu.make_async_copy(acc1_hbm.at[0, sl], a0buf, psem).wait()
        pltpu.make_async_copy(acc1_hbm.at[1, sl], a1buf, psem).wait()
        pltpu.make_async_copy(b1_hbm.at[0], bbuf, psem).wait()

        bb = bbuf[...]

        @pl.loop(0, RPT)
        def _(i):
            d = d0buf[i] + d1buf[i] + 1.0
            y = _rsqrt16(d)
            y2 = y * y
            s = y * (a0buf[i] + a1buf[i]) + y2 * hbuf[i] + bb
            z = jnp.maximum(s, 0.0)
            pbuf[i] = y * z
            qbuf[i] = y2 * z

        pltpu.sync_copy(pbuf, table.at[sl])

        @pl.when(cid == 0)
        def _():
            pltpu.sync_copy(qbuf, q_hbm.at[sl])

        @pl.loop(0, RPT)
        def _(i):
            pbuf[i] = jnp.zeros((HID,), jnp.float32)

        pltpu.sync_copy(pbuf, acc.at[sl])
        plsc.subcore_barrier()

        _agg_main_loop(table, sidx, didx, rows, acc, gsems, ssems)

        plsc.subcore_barrier()
        pltpu.sync_copy(acc.at[sl], out_hbm.at[cid, sl])

    return agg2_kernel(acc1, deg2, h1, b1r, src3, dst3)


def _tc_matmul1(x, w1t):
    def body(x_ref, w1t_ref, h1_ref):
        h = jnp.dot(x_ref[...], w1t_ref[...],
                    preferred_element_type=jnp.float32)
        h1_ref[...] = jnp.concatenate(
            [h, jnp.zeros((N_PAD - N, HID), jnp.float32)], axis=0)

    return pl.pallas_call(
        body,
        out_shape=jax.ShapeDtypeStruct((N_PAD, HID), jnp.float32),
    )(x, w1t)


def _tc_out(acc2, q, deg2, w2t, b2r):
    def body(agg_ref, q_ref, deg_ref, w2t_ref, b2_ref, out_ref):
        dis = lax.rsqrt(deg_ref[0] + deg_ref[1] + 1.0)
        m = dis * (agg_ref[0] + agg_ref[1]) + q_ref[...]
        out = jnp.dot(m[:N], w2t_ref[...],
                      preferred_element_type=jnp.float32) + b2_ref[...]
        out_ref[...] = jnp.maximum(out, 0.0)

    return pl.pallas_call(
        body,
        out_shape=jax.ShapeDtypeStruct((N, OUT_DIM), jnp.float32),
    )(acc2, q, deg2, w2t, b2r)


def _pk(a):
    return a.reshape(a.shape[:-2] + (a.shape[-2] * HID // 128, 128))


def _unpk(a):
    return a.reshape(a.shape[:-2] + (a.shape[-2] * 128 // HID, HID))


def kernel(x, edge_index, W1, b1, W2, b2):
    ei = edge_index.astype(jnp.int32)
    e = ei.shape[1]
    eflat = ei.reshape(2 * e)
    pad = TRASH + jnp.arange(E_PAD - e, dtype=jnp.int32) % (N_PAD - N)
    src3 = jnp.concatenate([eflat[:e], pad]).reshape(NW, NCH, CHUNK)
    dst3 = jnp.concatenate([eflat[e:], pad]).reshape(NW, NCH, CHUNK)
    w1t = W1.T
    w2t = W2.T

    deg2 = _sc_degree(dst3)
    h1 = _tc_matmul1(x, w1t)
    acc1 = _sc_agg1(h1, deg2, src3, dst3)
    acc2, q = _sc_agg2(acc1, deg2, h1, b1.reshape(1, HID), src3, dst3)
    return _tc_out(acc2, q, deg2, w2t, b2.reshape(1, OUT_DIM))


# speedup vs baseline: 1.0111x; 1.0003x over previous
import functools

import jax
import jax.numpy as jnp
from jax import lax
from jax.experimental import pallas as pl
from jax.experimental.pallas import tpu as pltpu
from jax.experimental.pallas import tpu_sc as plsc

N = 10000
IN_DIM = 128
HID = 16
OUT_DIM = 128
NC, NS = 2, 16
NW = NC * NS
N_PAD = 10240
TRASH = N
CHUNK = 128
NCH = 80
E_PAD = NW * NCH * CHUNK
NBUF = 8
RDIST = 4
RPT = N_PAD // NS


def _vector_mesh():
    return plsc.VectorSubcoreMesh(core_axis_name="c", subcore_axis_name="s")


_SC_PARAMS = pltpu.CompilerParams(use_tc_tiling_on_sc=False,
                                  needs_layout_passes=False)


def _rsqrt16(d):
    i = plsc.bitcast(d, jnp.int32)
    i = jnp.full((HID,), 0x5F3759DF, jnp.int32) - lax.shift_right_logical(i, 1)
    y = plsc.bitcast(i, jnp.float32)
    for _ in range(3):
        y = y * (1.5 - 0.5 * d * y * y)
    return y


def _sc_degree(dst3):
    @functools.partial(
        pl.kernel,
        out_type=jax.ShapeDtypeStruct((NC, N_PAD, HID), jnp.float32),
        mesh=_vector_mesh(),
        compiler_params=_SC_PARAMS,
        scratch_types=[
            pltpu.VMEM((NCH, CHUNK), jnp.int32),
            pltpu.VMEM((CHUNK, HID), jnp.float32),
            pltpu.VMEM((RPT, HID), jnp.float32),
            pltpu.VMEM_SHARED((N_PAD, HID), jnp.float32),
            pltpu.SemaphoreType.DMA,
        ],
    )
    def deg_kernel(dst_hbm, out_hbm, didx, ones_buf, zbuf, acc, ssem):
        cid = lax.axis_index("c")
        sid = lax.axis_index("s")
        wid = cid * NS + sid
        r0 = sid * RPT

        pltpu.sync_copy(dst_hbm.at[wid], didx)

        @pl.loop(0, RPT)
        def _(i):
            zbuf[i] = jnp.zeros((HID,), jnp.float32)

        @pl.loop(0, CHUNK)
        def _(i):
            ones_buf[i] = jnp.ones((HID,), jnp.float32)

        pltpu.sync_copy(zbuf, acc.at[pl.ds(r0, RPT)])
        plsc.subcore_barrier()

        @pl.loop(0, NCH)
        def _(c):
            pltpu.async_copy(ones_buf, acc.at[didx.at[c]], ssem, add=True)

        @pl.loop(0, NCH)
        def _(c):
            pltpu.make_async_copy(ones_buf, acc.at[didx.at[c]], ssem).wait()

        plsc.subcore_barrier()
        pltpu.sync_copy(acc.at[pl.ds(r0, RPT)],
                        out_hbm.at[cid, pl.ds(r0, RPT)])

    return deg_kernel(dst3)


def _agg_main_loop(table, sidx, didx, rows, acc, gsems, ssems):
    for j in range(NBUF):
        pltpu.async_copy(table.at[sidx.at[j]], rows.at[j], gsems.at[j])

    @pl.loop(0, NCH, step=NBUF)
    def _(g):
        for j in range(NBUF):
            c = g + j
            pltpu.make_async_copy(table.at[sidx.at[c]], rows.at[j],
                                  gsems.at[j]).wait()
            pltpu.async_copy(rows.at[j], acc.at[didx.at[c]],
                             ssems.at[j], add=True)
            cr = c - RDIST
            jr = (j - RDIST) % NBUF

            @pl.when(jnp.logical_and(cr >= 0, cr + NBUF < NCH))
            def _():
                pltpu.make_async_copy(rows.at[jr], acc.at[didx.at[cr]],
                                      ssems.at[jr]).wait()
                pltpu.async_copy(table.at[sidx.at[cr + NBUF]],
                                 rows.at[jr], gsems.at[jr])

    for j in range(NBUF):
        c = NCH - NBUF + j
        pltpu.make_async_copy(rows.at[j], acc.at[didx.at[c]],
                              ssems.at[j]).wait()


def _sc_agg1(h1, deg2, src3, dst3):
    @functools.partial(
        pl.kernel,
        out_type=jax.ShapeDtypeStruct((NC, N_PAD, HID), jnp.float32),
        mesh=_vector_mesh(),
        compiler_params=_SC_PARAMS,
        scratch_types=[
            pltpu.VMEM((NCH, CHUNK), jnp.int32),
            pltpu.VMEM((NCH, CHUNK), jnp.int32),
            pltpu.VMEM((NBUF, CHUNK, HID), jnp.float32),
            pltpu.VMEM((RPT, HID), jnp.float32),
            pltpu.VMEM((RPT, HID), jnp.float32),
            pltpu.VMEM((RPT, HID), jnp.float32),
            pltpu.VMEM((RPT, HID), jnp.float32),
            pltpu.VMEM_SHARED((N_PAD, HID), jnp.float32),
            pltpu.VMEM_SHARED((N_PAD, HID), jnp.float32),
            pltpu.SemaphoreType.DMA((NBUF,)),
            pltpu.SemaphoreType.DMA((NBUF,)),
            pltpu.SemaphoreType.DMA,
        ],
    )
    def agg1_kernel(h1_hbm, deg_hbm, src_hbm, dst_hbm, out_hbm,
                    sidx, didx, rows, hbuf, d0buf, d1buf, pbuf,
                    table, acc, gsems, ssems, psem):
        cid = lax.axis_index("c")
        sid = lax.axis_index("s")
        wid = cid * NS + sid
        r0 = sid * RPT
        sl = pl.ds(r0, RPT)

        pltpu.async_copy(src_hbm.at[wid], sidx, psem)
        pltpu.async_copy(dst_hbm.at[wid], didx, psem)
        pltpu.async_copy(h1_hbm.at[sl], hbuf, psem)
        pltpu.async_copy(deg_hbm.at[0, sl], d0buf, psem)
        pltpu.async_copy(deg_hbm.at[1, sl], d1buf, psem)
        pltpu.make_async_copy(src_hbm.at[wid], sidx, psem).wait()
        pltpu.make_async_copy(dst_hbm.at[wid], didx, psem).wait()
        pltpu.make_async_copy(h1_hbm.at[sl], hbuf, psem).wait()
        pltpu.make_async_copy(deg_hbm.at[0, sl], d0buf, psem).wait()
        pltpu.make_async_copy(deg_hbm.at[1, sl], d1buf, psem).wait()

        @pl.loop(0, RPT)
        def _(i):
            d = d0buf[i] + d1buf[i] + 1.0
            pbuf[i] = _rsqrt16(d) * hbuf[i]

        pltpu.sync_copy(pbuf, table.at[sl])

        @pl.loop(0, RPT)
        def _(i):
            pbuf[i] = jnp.zeros((HID,), jnp.float32)

        pltpu.sync_copy(pbuf, acc.at[sl])
        plsc.subcore_barrier()

        _agg_main_loop(table, sidx, didx, rows, acc, gsems, ssems)

        plsc.subcore_barrier()
        pltpu.sync_copy(acc.at[sl], out_hbm.at[cid, sl])

    return agg1_kernel(h1, deg2, src3, dst3)


def _sc_agg2(acc1, deg2, h1, b1r, src3, dst3):
    @functools.partial(
        pl.kernel,
        out_type=(
            jax.ShapeDtypeStruct((NC, N_PAD, HID), jnp.float32),
            jax.ShapeDtypeStruct((N_PAD, HID), jnp.float32),
        ),
        mesh=_vector_mesh(),
        compiler_params=_SC_PARAMS,
        scratch_types=[
            pltpu.VMEM((NCH, CHUNK), jnp.int32),
            pltpu.VMEM((NCH, CHUNK), jnp.int32),
            pltpu.VMEM((NBUF, CHUNK, HID), jnp.float32),
            pltpu.VMEM((RPT, HID), jnp.float32),
            pltpu.VMEM((RPT, HID), jnp.float32),
            pltpu.VMEM((RPT, HID), jnp.float32),
            pltpu.VMEM((RPT, HID), jnp.float32),
            pltpu.VMEM((RPT, HID), jnp.float32),
            pltpu.VMEM((RPT, HID), jnp.float32),
            pltpu.VMEM((RPT, HID), jnp.float32),
            pltpu.VMEM((HID,), jnp.float32),
            pltpu.VMEM_SHARED((N_PAD, HID), jnp.float32),
            pltpu.VMEM_SHARED((N_PAD, HID), jnp.float32),
            pltpu.SemaphoreType.DMA((NBUF,)),
            pltpu.SemaphoreType.DMA((NBUF,)),
            pltpu.SemaphoreType.DMA,
        ],
    )
    def agg2_kernel(acc1_hbm, deg_hbm, h1_hbm, b1_hbm, src_hbm, dst_hbm,
                    out_hbm, q_hbm,
                    sidx, didx, rows, hbuf, d0buf, d1buf, a0buf, a1buf,
                    pbuf, qbuf, bbuf, table, acc, gsems, ssems, psem):
        cid = lax.axis_index("c")
        sid = lax.axis_index("s")
        wid = cid * NS + sid
        r0 = sid * RPT
        sl = pl.ds(r0, RPT)

        pltpu.async_copy(src_hbm.at[wid], sidx, psem)
        pltpu.async_copy(dst_hbm.at[wid], didx, psem)
        pltpu.async_copy(h1_hbm.at[sl], hbuf, psem)
        pltpu.async_copy(deg_hbm.at[0, sl], d0buf, psem)
        pltpu.async_copy(deg_hbm.at[1, sl], d1buf, psem)
        pltpu.async_copy(acc1_hbm.at[0, sl], a0buf, psem)
        pltpu.async_copy(acc1_hbm.at[1, sl], a1buf, psem)
        pltpu.async_copy(b1_hbm.at[0], bbuf, psem)
        pltpu.make_async_copy(src_hbm.at[wid], sidx, psem).wait()
        pltpu.make_async_copy(dst_hbm.at[wid], didx, psem).wait()
        pltpu.make_async_copy(h1_hbm.at[sl], hbuf, psem).wait()
        pltpu.make_async_copy(deg_hbm.at[0, sl], d0buf, psem).wait()
        pltpu.make_async_copy(deg_hbm.at[1, sl], d1buf, psem).wait()
        pltpu.make_async_copy(acc1_hbm.at[0, sl], a0buf, psem).wait()
        pltpu.make_async_copy(acc1_hbm.at[1, sl], a1buf, psem).wait()
        pltpu.make_async_copy(b1_hbm.at[0], bbuf, psem).wait()

        bb = bbuf[...]

        @pl.loop(0, RPT)
        def _(i):
            d = d0buf[i] + d1buf[i] + 1.0
            y = _rsqrt16(d)
            y2 = y * y
            s = y * (a0buf[i] + a1buf[i]) + y2 * hbuf[i] + bb
            z = jnp.maximum(s, 0.0)
            pbuf[i] = y * z
            qbuf[i] = y2 * z

        pltpu.sync_copy(pbuf, table.at[sl])

        @pl.when(cid == 0)
        def _():
            pltpu.sync_copy(qbuf, q_hbm.at[sl])

        @pl.loop(0, RPT)
        def _(i):
            pbuf[i] = jnp.zeros((HID,), jnp.float32)

        pltpu.sync_copy(pbuf, acc.at[sl])
        plsc.subcore_barrier()

        _agg_main_loop(table, sidx, didx, rows, acc, gsems, ssems)

        plsc.subcore_barrier()
        pltpu.sync_copy(acc.at[sl], out_hbm.at[cid, sl])

    return agg2_kernel(acc1, deg2, h1, b1r, src3, dst3)


def _tc_matmul1(x, w1t):
    def body(x_ref, w1t_ref, h1_ref):
        h = jnp.dot(x_ref[...], w1t_ref[...],
                    preferred_element_type=jnp.float32)
        h1_ref[...] = jnp.concatenate(
            [h, jnp.zeros((N_PAD - N, HID), jnp.float32)], axis=0)

    return pl.pallas_call(
        body,
        out_shape=jax.ShapeDtypeStruct((N_PAD, HID), jnp.float32),
    )(x, w1t)


def _tc_out(acc2, q, deg2, w2t, b2r):
    def body(agg_ref, q_ref, deg_ref, w2t_ref, b2_ref, out_ref):
        dis = lax.rsqrt(deg_ref[0] + deg_ref[1] + 1.0)
        m = dis * (agg_ref[0] + agg_ref[1]) + q_ref[...]
        out = jnp.dot(m[:N], w2t_ref[...],
                      preferred_element_type=jnp.float32) + b2_ref[...]
        out_ref[...] = jnp.maximum(out, 0.0)

    return pl.pallas_call(
        body,
        out_shape=jax.ShapeDtypeStruct((N, OUT_DIM), jnp.float32),
    )(acc2, q, deg2, w2t, b2r)


def kernel(x, edge_index, W1, b1, W2, b2):
    ei = edge_index.astype(jnp.int32)
    e = ei.shape[1]
    eflat = ei.reshape(2 * e)
    pad = TRASH + jnp.arange(E_PAD - e, dtype=jnp.int32) % (N_PAD - N)
    src3 = jnp.concatenate([eflat[:e], pad]).reshape(NW, NCH, CHUNK)
    dst3 = jnp.concatenate([eflat[e:], pad]).reshape(NW, NCH, CHUNK)
    w1t = W1.T
    w2t = W2.T

    deg2 = _sc_degree(dst3)
    h1 = _tc_matmul1(x, w1t)
    acc1 = _sc_agg1(h1, deg2, src3, dst3)
    acc2, q = _sc_agg2(acc1, deg2, h1, b1.reshape(1, HID), src3, dst3)
    return _tc_out(acc2, q, deg2, w2t, b2.reshape(1, OUT_DIM))
